# Initial kernel scaffold; baseline (speedup 1.0000x reference)
import jax
import jax.numpy as jnp
from jax.experimental import pallas as pl


def kernel(indices, embeddings):
    raise NotImplementedError("write your pallas kernel here")


# speedup vs baseline: 4.2992x; 4.2992x over previous
import functools

import jax
import jax.numpy as jnp
from jax import lax
from jax.experimental import pallas as pl
from jax.experimental.pallas import tpu as pltpu
from jax.experimental.pallas import tpu_sc as plsc

ROWS_PER_GATHER = 128
NBUF = 8
PREFETCH = 4


@functools.lru_cache(maxsize=None)
def _make_sc_gather(n_rows: int, embed_num: int, embed_dim: int):
    info = plsc.get_sparse_core_info()
    nc, ns = info.num_cores, info.num_subcores
    nw = nc * ns
    rows_per_w = n_rows // nw
    chunks = rows_per_w // ROWS_PER_GATHER
    rpg = ROWS_PER_GATHER

    mesh = plsc.VectorSubcoreMesh(core_axis_name="c", subcore_axis_name="s")

    @functools.partial(
        pl.kernel,
        mesh=mesh,
        out_type=jax.ShapeDtypeStruct((n_rows, embed_dim), jnp.float32),
        scratch_types=[
            pltpu.VMEM((chunks, rpg), jnp.int32),
            pltpu.VMEM((NBUF, rpg, embed_dim), jnp.float32),
            pltpu.SemaphoreType.DMA((NBUF,)),
            pltpu.SemaphoreType.DMA((NBUF,)),
        ],
        compiler_params=pltpu.CompilerParams(use_tc_tiling_on_sc=False),
    )
    def k(idx_hbm, table_hbm, out_hbm, idx_v, rows_v, gsems, osems):
        wid = lax.axis_index("s") * nc + lax.axis_index("c")
        pltpu.sync_copy(idx_hbm.at[wid], idx_v)
        base = wid * rows_per_w

        for b in range(PREFETCH):
            pltpu.async_copy(table_hbm.at[idx_v.at[b]], rows_v.at[b], gsems.at[b])

        def step(j, carry):
            b = lax.rem(j, NBUF)
            b2 = lax.rem(j + PREFETCH, NBUF)

            pltpu.make_async_copy(
                table_hbm.at[idx_v.at[j]], rows_v.at[b], gsems.at[b]
            ).wait()

            @pl.when(j + PREFETCH >= NBUF)
            def _():
                pltpu.make_async_copy(
                    rows_v.at[b2],
                    out_hbm.at[pl.ds(base + (j + PREFETCH - NBUF) * rpg, rpg)],
                    osems.at[b2],
                ).wait()

            @pl.when(j + PREFETCH < chunks)
            def _():
                pltpu.async_copy(
                    table_hbm.at[idx_v.at[j + PREFETCH]], rows_v.at[b2], gsems.at[b2]
                )

            pltpu.async_copy(
                rows_v.at[b],
                out_hbm.at[pl.ds(base + j * rpg, rpg)],
                osems.at[b],
            )
            return carry

        lax.fori_loop(0, chunks, step, 0)

        for b in range(PREFETCH):
            j = chunks - PREFETCH + b
            pltpu.make_async_copy(
                rows_v.at[j % NBUF],
                out_hbm.at[pl.ds(base + j * rpg, rpg)],
                osems.at[j % NBUF],
            ).wait()

    return k, nw, chunks


def kernel(indices, embeddings):
    b, tok = indices.shape
    embed_num, embed_dim = embeddings.shape
    n_rows = b * tok
    k, nw, chunks = _make_sc_gather(n_rows, embed_num, embed_dim)
    idx3 = indices.reshape(nw, chunks, ROWS_PER_GATHER)
    out = k(idx3, embeddings)
    return out.reshape(b, tok, embed_dim)


# speedup vs baseline: 4.7913x; 1.1145x over previous
import functools

import jax
import jax.numpy as jnp
from jax import lax
from jax.experimental import pallas as pl
from jax.experimental.pallas import tpu as pltpu
from jax.experimental.pallas import tpu_sc as plsc

ROWS_PER_GATHER = 128
NBUF = 8
PREFETCH = 4


@functools.lru_cache(maxsize=None)
def _make_sc_gather(n_rows: int, embed_num: int, embed_dim: int):
    info = plsc.get_sparse_core_info()
    nc, ns = info.num_cores, info.num_subcores
    nw = nc * ns
    rows_per_w = n_rows // nw
    chunks = rows_per_w // ROWS_PER_GATHER
    rpg = ROWS_PER_GATHER

    mesh = plsc.VectorSubcoreMesh(core_axis_name="c", subcore_axis_name="s")

    @functools.partial(
        pl.kernel,
        mesh=mesh,
        out_type=jax.ShapeDtypeStruct((n_rows, embed_dim), jnp.float32),
        scratch_types=[
            pltpu.VMEM((chunks, rpg), jnp.int32),
            pltpu.VMEM((NBUF, rpg, embed_dim), jnp.float32),
            pltpu.VMEM_SHARED((embed_num, embed_dim), jnp.float32),
            pltpu.SemaphoreType.DMA((NBUF,)),
            pltpu.SemaphoreType.DMA((NBUF,)),
        ],
        compiler_params=pltpu.CompilerParams(use_tc_tiling_on_sc=False),
    )
    def k(idx_hbm, table_hbm, out_hbm, idx_v, rows_v, table_sh, gsems, osems):
        sid = lax.axis_index("s")
        wid = sid * nc + lax.axis_index("c")
        tslice = embed_num // ns
        pltpu.sync_copy(
            table_hbm.at[pl.ds(sid * tslice, tslice)],
            table_sh.at[pl.ds(sid * tslice, tslice)],
        )
        pltpu.sync_copy(idx_hbm.at[wid], idx_v)
        plsc.subcore_barrier()
        base = wid * rows_per_w

        for b in range(PREFETCH):
            pltpu.async_copy(table_sh.at[idx_v.at[b]], rows_v.at[b], gsems.at[b])

        def step(j, carry):
            b = lax.rem(j, NBUF)
            b2 = lax.rem(j + PREFETCH, NBUF)

            pltpu.make_async_copy(
                table_sh.at[idx_v.at[j]], rows_v.at[b], gsems.at[b]
            ).wait()

            @pl.when(j + PREFETCH >= NBUF)
            def _():
                pltpu.make_async_copy(
                    rows_v.at[b2],
                    out_hbm.at[pl.ds(base + (j + PREFETCH - NBUF) * rpg, rpg)],
                    osems.at[b2],
                ).wait()

            @pl.when(j + PREFETCH < chunks)
            def _():
                pltpu.async_copy(
                    table_sh.at[idx_v.at[j + PREFETCH]], rows_v.at[b2], gsems.at[b2]
                )

            pltpu.async_copy(
                rows_v.at[b],
                out_hbm.at[pl.ds(base + j * rpg, rpg)],
                osems.at[b],
            )
            return carry

        lax.fori_loop(0, chunks, step, 0)

        for b in range(PREFETCH):
            j = chunks - PREFETCH + b
            pltpu.make_async_copy(
                rows_v.at[j % NBUF],
                out_hbm.at[pl.ds(base + j * rpg, rpg)],
                osems.at[j % NBUF],
            ).wait()

    return k, nw, chunks


def kernel(indices, embeddings):
    b, tok = indices.shape
    embed_num, embed_dim = embeddings.shape
    n_rows = b * tok
    k, nw, chunks = _make_sc_gather(n_rows, embed_num, embed_dim)
    idx3 = indices.reshape(nw, chunks, ROWS_PER_GATHER)
    out = k(idx3, embeddings)
    return out.reshape(b, tok, embed_dim)
